# baseline (speedup 1.0000x reference)
import functools

import jax
import jax.numpy as jnp
from jax import lax
from jax.experimental import pallas as pl
from jax.experimental.pallas import tpu as pltpu
from jax.experimental.pallas import tpu_sc as plsc

S, B, D = 2048, 4, 2048
NW = 32
NTILE = 16
S_PER_W = S // (NW // B)

F = 24
NSLOT = 12
NBAT = F // NSLOT
S_STREAM = S_PER_W - F

K = 12
CK = [8, 8] + [12] * 18
NCH = len(CK)
CS = [sum(CK[:i]) for i in range(NCH)]

_mesh = plsc.VectorSubcoreMesh(core_axis_name="c", subcore_axis_name="s")


@functools.partial(
    pl.kernel,
    mesh=_mesh,
    out_type=jax.ShapeDtypeStruct((S, B, D), jnp.float32),
    scratch_types=[
        pltpu.VMEM((S_PER_W,), jnp.int32),
        pltpu.VMEM((NCH, 16), jnp.int32),
        pltpu.VMEM((K, 1, D), jnp.float32),
        pltpu.VMEM((K, 1, D), jnp.float32),
        pltpu.VMEM((K, 1, D), jnp.float32),
        pltpu.SMEM((F,), jnp.int32),
        pltpu.VMEM_SHARED((NTILE, F), jnp.int32),
        pltpu.VMEM_SHARED((NTILE, F, 1, D), jnp.float32),
        pltpu.SemaphoreType.DMA,
        pltpu.SemaphoreType.DMA,
        pltpu.SemaphoreType.DMA,
        pltpu.SemaphoreType.DMA,
        pltpu.SemaphoreType.DMA,
        pltpu.SemaphoreType.DMA,
        pltpu.SemaphoreType.DMA,
        pltpu.SemaphoreType.DMA,
    ],
)
def _sc_gather(perm_hbm, srcf_hbm, out_hbm, perm_v, idx2d, buf0, buf1, buf2,
               idx_sm, idx_sp, sp, sg0, sg1, sg2, ss0, ss1, ss2, dg, dsme):
    cid = lax.axis_index("c")
    sid = lax.axis_index("s")
    wid = sid * 2 + cid
    b = wid % B
    s0 = (wid // B) * S_PER_W

    pltpu.sync_copy(perm_hbm.at[b, pl.ds(s0, S_PER_W)], perm_v)

    pltpu.sync_copy(perm_v.at[pl.ds(S_STREAM, F)], idx_sp.at[sid])
    pltpu.sync_copy(idx_sp.at[sid], idx_sm)

    for i in range(NCH):
        idx2d[i, pl.ds(0, 16)] = perm_v[pl.ds(CS[i], 16)]

    bufs = (buf0, buf1, buf2)
    gsems = (sg0, sg1, sg2)
    ssems = (ss0, ss1, ss2)

    def gather(c):
        p = c % 3
        return pltpu.make_async_copy(
            srcf_hbm.at[idx2d.at[c, pl.ds(0, CK[c])], pl.ds(b, 1)],
            bufs[p].at[pl.ds(0, CK[c])], gsems[p])

    def store(c):
        p = c % 3
        return pltpu.make_async_copy(
            bufs[p].at[pl.ds(0, CK[c])],
            out_hbm.at[pl.ds(s0 + CS[c], CK[c]), pl.ds(b, 1)], ssems[p])

    def drows(m):
        def body(i, _):
            j = m * NSLOT + i
            p = idx_sm[j]
            pltpu.make_async_copy(
                srcf_hbm.at[pl.ds(p, 1), pl.ds(b, 1)],
                sp.at[sid, pl.ds(j, 1)], dg).start()
            return 0
        lax.fori_loop(0, NSLOT, body, 0)

    def dwait(m):
        pltpu.make_async_copy(
            srcf_hbm.at[pl.ds(0, NSLOT), pl.ds(b, 1)],
            sp.at[sid, pl.ds(m * NSLOT, NSLOT)], dg).wait()

    def dstore(m):
        return pltpu.make_async_copy(
            sp.at[sid, pl.ds(m * NSLOT, NSLOT)],
            out_hbm.at[pl.ds(s0 + S_STREAM + m * NSLOT, NSLOT), pl.ds(b, 1)],
            dsme)

    drows(0)
    drows(1)
    gather(0).start()
    gather(1).start()
    for c in range(NCH):
        g = c + 2
        if g < NCH:
            if g >= 3:
                store(g - 3).wait()
            gather(g).start()
        if c == 7:
            dwait(0)
            dstore(0).start()
        if c == 11:
            dwait(1)
            dstore(1).start()
        gather(c).wait()
        store(c).start()
    store(NCH - 3).wait()
    store(NCH - 2).wait()
    store(NCH - 1).wait()
    dstore(0).wait()
    dstore(1).wait()


def kernel(permute, src):
    return _sc_gather(permute, src)


# speedup vs baseline: 1.0175x; 1.0175x over previous
import functools

import jax
import jax.numpy as jnp
from jax import lax
from jax.experimental import pallas as pl
from jax.experimental.pallas import tpu as pltpu
from jax.experimental.pallas import tpu_sc as plsc

S, B, D = 2048, 4, 2048
NW = 32
S_PER_W = S // (NW // B)
K = 16
CK = [8, 8] + [16] * 15
NCH = len(CK)
CS = [sum(CK[:i]) for i in range(NCH)]

_mesh = plsc.VectorSubcoreMesh(core_axis_name="c", subcore_axis_name="s")


@functools.partial(
    pl.kernel,
    mesh=_mesh,
    out_type=jax.ShapeDtypeStruct((S, B, D), jnp.float32),
    scratch_types=[
        pltpu.VMEM((S_PER_W,), jnp.int32),
        pltpu.VMEM((NCH, 32), jnp.int32),
        pltpu.VMEM((K, 1, D), jnp.float32),
        pltpu.VMEM((K, 1, D), jnp.float32),
        pltpu.VMEM((K, 1, D), jnp.float32),
        pltpu.SemaphoreType.DMA,
        pltpu.SemaphoreType.DMA,
        pltpu.SemaphoreType.DMA,
        pltpu.SemaphoreType.DMA,
        pltpu.SemaphoreType.DMA,
        pltpu.SemaphoreType.DMA,
    ],
)
def _sc_gather(perm_hbm, srcf_hbm, out_hbm, perm_v, idx2d, buf0, buf1, buf2,
               sg0, sg1, sg2, ss0, ss1, ss2):
    cid = lax.axis_index("c")
    sid = lax.axis_index("s")
    wid = sid * 2 + cid
    b = wid % B
    s0 = (wid // B) * S_PER_W

    pltpu.sync_copy(perm_hbm.at[b, pl.ds(s0, S_PER_W)], perm_v)

    for i in range(NCH):
        for j in (0, 16):
            if j < CK[i] and CS[i] + j + 16 <= S_PER_W:
                idx2d[i, pl.ds(j, 16)] = perm_v[pl.ds(CS[i] + j, 16)]

    bufs = (buf0, buf1, buf2)
    gsems = (sg0, sg1, sg2)
    ssems = (ss0, ss1, ss2)

    def gather(c):
        p = c % 3
        return pltpu.make_async_copy(
            srcf_hbm.at[idx2d.at[c, pl.ds(0, CK[c])], pl.ds(b, 1)],
            bufs[p].at[pl.ds(0, CK[c])], gsems[p])

    def store(c):
        p = c % 3
        return pltpu.make_async_copy(
            bufs[p].at[pl.ds(0, CK[c])],
            out_hbm.at[pl.ds(s0 + CS[c], CK[c]), pl.ds(b, 1)], ssems[p])

    gather(0).start()
    gather(1).start()
    for c in range(NCH):
        g = c + 2
        if g < NCH:
            if g >= 3:
                store(g - 3).wait()
            gather(g).start()
        gather(c).wait()
        store(c).start()
    store(NCH - 3).wait()
    store(NCH - 2).wait()
    store(NCH - 1).wait()


def kernel(permute, src):
    return _sc_gather(permute, src)
